# baseline (device time: 24713 ns/iter reference)
import jax
import jax.numpy as jnp
from jax import lax
from jax.experimental import pallas as pl
from jax.experimental.pallas import tpu as pltpu

N_DEV = 32
N_CHUNK = 2


def kernel(x):
    m_per, n_per = x.shape
    H = m_per // N_CHUNK

    def body(x_ref, o_ref, maxs_ref, sums_ref,
             send_m, recv_m, send_s, recv_s):
        me = lax.axis_index("i")

        barrier_sem = pltpu.get_barrier_semaphore()
        for k in range(1, N_DEV):
            pl.semaphore_signal(
                barrier_sem, inc=1,
                device_id=((me + k) % N_DEV,),
                device_id_type=pl.DeviceIdType.MESH,
            )

        xv = x_ref[:, :]
        m_loc = jnp.max(xv, axis=1)
        maxs_ref[me, :] = m_loc

        e0 = jnp.exp(xv[:H, :] - m_loc[:H, None])
        sums_ref[0, me, :] = jnp.sum(e0, axis=1)
        o_ref[pl.ds(0, H), :] = e0

        pl.semaphore_wait(barrier_sem, N_DEV - 1)

        def broadcast(src, dst, ssems, rsems):
            rdmas = []
            for k in range(1, N_DEV):
                rdma = pltpu.make_async_remote_copy(
                    src_ref=src,
                    dst_ref=dst,
                    send_sem=ssems.at[k - 1],
                    recv_sem=rsems.at[k - 1],
                    device_id=((me + k) % N_DEV,),
                    device_id_type=pl.DeviceIdType.MESH,
                )
                rdma.start()
                rdmas.append(rdma)
            return rdmas

        m_rdmas = broadcast(maxs_ref.at[me], maxs_ref.at[me],
                            send_m, recv_m)
        s0_rdmas = broadcast(sums_ref.at[0, me], sums_ref.at[0, me],
                             send_s.at[0], recv_s.at[0])

        e1 = jnp.exp(xv[H:, :] - m_loc[H:, None])
        sums_ref[1, me, :] = jnp.sum(e1, axis=1)
        o_ref[pl.ds(H, H), :] = e1

        s1_rdmas = broadcast(sums_ref.at[1, me], sums_ref.at[1, me],
                             send_s.at[1], recv_s.at[1])

        for rdma in m_rdmas:
            rdma.wait_recv()
        all_m = maxs_ref[:, :]
        m_g = jnp.max(all_m, axis=0)
        w = jnp.exp(all_m - m_g[None, :])
        corr = jnp.exp(m_loc - m_g)

        for rdma in s0_rdmas:
            rdma.wait_recv()
        s_g0 = jnp.sum(sums_ref[0, :, :] * w[:, :H], axis=0)
        scale0 = corr[:H] / s_g0
        o_ref[pl.ds(0, H), :] = o_ref[pl.ds(0, H), :] * scale0[:, None]

        for rdma in s1_rdmas:
            rdma.wait_recv()
        s_g1 = jnp.sum(sums_ref[1, :, :] * w[:, H:], axis=0)
        scale1 = corr[H:] / s_g1
        o_ref[pl.ds(H, H), :] = o_ref[pl.ds(H, H), :] * scale1[:, None]

        for rdma in m_rdmas + s0_rdmas + s1_rdmas:
            rdma.wait_send()

    return pl.pallas_call(
        body,
        out_shape=jax.ShapeDtypeStruct((m_per, n_per), jnp.float32),
        in_specs=[pl.BlockSpec(memory_space=pltpu.VMEM)],
        out_specs=pl.BlockSpec(memory_space=pltpu.VMEM),
        scratch_shapes=[
            pltpu.VMEM((N_DEV, m_per), jnp.float32),
            pltpu.VMEM((N_CHUNK, N_DEV, H), jnp.float32),
            pltpu.SemaphoreType.DMA((N_DEV - 1,)),
            pltpu.SemaphoreType.DMA((N_DEV - 1,)),
            pltpu.SemaphoreType.DMA((N_CHUNK, N_DEV - 1)),
            pltpu.SemaphoreType.DMA((N_CHUNK, N_DEV - 1)),
        ],
        compiler_params=pltpu.CompilerParams(collective_id=0),
    )(x)


# device time: 24398 ns/iter; 1.0129x vs baseline; 1.0129x over previous
import jax
import jax.numpy as jnp
from jax import lax
from jax.experimental import pallas as pl
from jax.experimental.pallas import tpu as pltpu

N_DEV = 32
COL_TILE = 256


def kernel(x):
    m_per, n_per = x.shape

    def body(x_ref, o_ref, stats_ref, send_sems, recv_sems):
        me = lax.axis_index("i")

        barrier_sem = pltpu.get_barrier_semaphore()
        for k in range(1, N_DEV):
            pl.semaphore_signal(
                barrier_sem, inc=1,
                device_id=((me + k) % N_DEV,),
                device_id_type=pl.DeviceIdType.MESH,
            )

        xv = x_ref[:, :]
        m_run = jnp.max(xv[:, :COL_TILE], axis=1)
        s_run = jnp.sum(jnp.exp(xv[:, :COL_TILE] - m_run[:, None]), axis=1)
        for t in range(1, n_per // COL_TILE):
            tile = xv[:, t * COL_TILE:(t + 1) * COL_TILE]
            m_new = jnp.maximum(m_run, jnp.max(tile, axis=1))
            s_run = s_run * jnp.exp(m_run - m_new) + jnp.sum(
                jnp.exp(tile - m_new[:, None]), axis=1)
            m_run = m_new
        stats_ref[me, pl.ds(0, m_per)] = m_run
        stats_ref[me, pl.ds(m_per, m_per)] = s_run

        pl.semaphore_wait(barrier_sem, N_DEV - 1)

        rdmas = []
        for k in range(1, N_DEV):
            rdma = pltpu.make_async_remote_copy(
                src_ref=stats_ref.at[me],
                dst_ref=stats_ref.at[me],
                send_sem=send_sems.at[k - 1],
                recv_sem=recv_sems.at[k - 1],
                device_id=((me + k) % N_DEV,),
                device_id_type=pl.DeviceIdType.MESH,
            )
            rdma.start()
            rdmas.append(rdma)

        for rdma in rdmas:
            rdma.wait_recv()

        all_m = stats_ref[:, pl.ds(0, m_per)]
        all_s = stats_ref[:, pl.ds(m_per, m_per)]
        m_g = jnp.max(all_m, axis=0)
        s_g = jnp.sum(all_s * jnp.exp(all_m - m_g[None, :]), axis=0)
        r = 1.0 / s_g
        o_ref[:, :] = jnp.exp(xv - m_g[:, None]) * r[:, None]

        for rdma in rdmas:
            rdma.wait_send()

    return pl.pallas_call(
        body,
        out_shape=jax.ShapeDtypeStruct((m_per, n_per), jnp.float32),
        in_specs=[pl.BlockSpec(memory_space=pltpu.VMEM)],
        out_specs=pl.BlockSpec(memory_space=pltpu.VMEM),
        scratch_shapes=[
            pltpu.VMEM((N_DEV, 2 * m_per), jnp.float32),
            pltpu.SemaphoreType.DMA((N_DEV - 1,)),
            pltpu.SemaphoreType.DMA((N_DEV - 1,)),
        ],
        compiler_params=pltpu.CompilerParams(collective_id=0),
    )(x)
